# baseline (speedup 1.0000x reference)
import functools

import jax
import jax.numpy as jnp
from jax import lax
from jax.experimental import pallas as pl
from jax.experimental.pallas import tpu as pltpu
from jax.experimental.pallas import tpu_sc as plsc


def _sc_geometry():
    try:
        info = plsc.get_sparse_core_info()
        return info.num_cores, info.num_subcores, info.num_lanes
    except Exception:
        return 2, 16, 16


def kernel(token_ids, idf, W, b):
    Bn, Ln = token_ids.shape
    Vn, Dn = W.shape
    NC, NS, LANES = _sc_geometry()
    NW = NC * NS
    assert Bn % NW == 0
    docs_per_w = Bn // NW
    assert docs_per_w % 4 == 0 and docs_per_w >= 8
    assert Ln % 8 == 0
    n_chunks = Dn // LANES
    c0 = min(Ln, 104)
    c1 = Ln - c0
    Ln_pad = ((Ln + LANES - 1) // LANES) * LANES
    n_groups = Ln_pad // LANES

    mesh = plsc.VectorSubcoreMesh(core_axis_name="c", subcore_axis_name="s")

    @functools.partial(
        pl.kernel,
        mesh=mesh,
        out_type=[
            jax.ShapeDtypeStruct((Bn, Dn), jnp.float32),
            jax.ShapeDtypeStruct((Bn, LANES), jnp.float32)
        ],
        scratch_types=[
            pltpu.VMEM((Ln,), jnp.int32),
            pltpu.VMEM((Ln,), jnp.int32),
            pltpu.VMEM((Ln,), jnp.int32),
            pltpu.VMEM((Ln,), jnp.int32),
            pltpu.VMEM((Ln_pad,), jnp.float32),
            pltpu.VMEM((Ln_pad,), jnp.float32),
            pltpu.VMEM((Ln_pad, Dn), jnp.float32),
            pltpu.VMEM((Ln_pad, Dn), jnp.float32),
            pltpu.VMEM((Dn,), jnp.float32),
            pltpu.VMEM((docs_per_w, Dn), jnp.float32),
            pltpu.VMEM((docs_per_w, LANES), jnp.float32),
            pltpu.SemaphoreType.DMA,
            pltpu.SemaphoreType.DMA,
            pltpu.SemaphoreType.DMA,
            pltpu.SemaphoreType.DMA,
            pltpu.SemaphoreType.DMA,
            pltpu.SemaphoreType.DMA,
        ],
    )
    def sc_kernel(tok_hbm, idf_hbm, w_hbm, b_hbm, emb_hbm, abs_hbm,
                  t0, t1, t2, t3, idf0, idf1, rows0, rows1, b_v, emb_v,
                  abs_v, sem0, sem1, st0, st1, st2, st3):
        wid = lax.axis_index("s") * NC + lax.axis_index("c")
        base = wid * docs_per_w
        toks = ((t0, st0), (t1, st1), (t2, st2), (t3, st3))
        bufs = ((idf0, rows0, sem0), (idf1, rows1, sem1))
        inv_l = 1.0 / float(Ln)

        def tok_issue(d, k):
            row = base + jnp.minimum(d, docs_per_w - 1)
            tb, st = toks[k]
            pltpu.async_copy(tok_hbm.at[row], tb, st)

        def tok_wait(k):
            tb, st = toks[k]
            pltpu.make_async_copy(tok_hbm.at[0], tb, st).wait()

        pltpu.sync_copy(b_hbm, b_v)

        zvec = jnp.zeros((LANES,), jnp.float32)
        if Ln_pad > Ln:
            g_last = Ln_pad - LANES
            for idfb, rowb, _ in bufs:
                idfb[pl.ds(g_last, LANES)] = zvec
                for r in range(Ln, Ln_pad):
                    for c in range(n_chunks):
                        rowb[r, pl.ds(c * LANES, LANES)] = zvec

        def issue(d, buf, k):
            idfb, rowb, sem = buf
            tb, _ = toks[k]
            i1 = tb.at[pl.ds(0, c0)]
            i2 = tb.at[pl.ds(c0, c1)]
            pltpu.async_copy(idf_hbm.at[i1], idfb.at[pl.ds(0, c0)], sem)
            pltpu.async_copy(idf_hbm.at[i2], idfb.at[pl.ds(c0, c1)], sem)
            pltpu.async_copy(w_hbm.at[i1], rowb.at[pl.ds(0, c0)], sem)
            pltpu.async_copy(w_hbm.at[i2], rowb.at[pl.ds(c0, c1)], sem)

        def wait(buf):
            idfb, rowb, sem = buf
            pltpu.make_async_copy(idf_hbm.at[pl.ds(0, c0)],
                                  idfb.at[pl.ds(0, c0)], sem).wait()
            pltpu.make_async_copy(idf_hbm.at[pl.ds(0, c1)],
                                  idfb.at[pl.ds(c0, c1)], sem).wait()
            pltpu.make_async_copy(w_hbm.at[pl.ds(0, c0)],
                                  rowb.at[pl.ds(0, c0)], sem).wait()
            pltpu.make_async_copy(w_hbm.at[pl.ds(0, c1)],
                                  rowb.at[pl.ds(c0, c1)], sem).wait()

        def compute(d, buf):
            idfb, rowb, _ = buf

            def grp_body(g, accs):
                tok0 = g * LANES
                wv = idfb[pl.ds(tok0, LANES)]
                for j in range(LANES):
                    wgt = wv[j]
                    accs = tuple(
                        accs[c] + wgt * rowb[tok0 + j,
                                             pl.ds(c * LANES, LANES)]
                        for c in range(n_chunks)
                    )
                return accs

            accs0 = tuple(jnp.zeros((LANES,), jnp.float32)
                          for _ in range(n_chunks))
            accs = lax.fori_loop(0, n_groups, grp_body, accs0)

            abssum = jnp.zeros((LANES,), jnp.float32)
            for c in range(n_chunks):
                e = accs[c] * inv_l + b_v[pl.ds(c * LANES, LANES)]
                emb_v[d, pl.ds(c * LANES, LANES)] = e
                abssum = abssum + jnp.abs(e)
            abs_v[d] = abssum

        for k in range(4):
            tok_issue(k, k)
        tok_wait(0)
        issue(0, bufs[0], 0)
        tok_wait(1)
        issue(1, bufs[1], 1)

        def pipe_body(g, carry):
            for j in range(4):
                d = 4 * g + j
                wait(bufs[j % 2])
                compute(d, bufs[j % 2])
                tok_issue(d + 4, j)
                tok_wait((j + 2) % 4)
                issue(d + 2, bufs[j % 2], (j + 2) % 4)
            return carry

        lax.fori_loop(0, docs_per_w // 4 - 1, pipe_body, 0)

        nd = docs_per_w
        wait(bufs[0])
        compute(nd - 4, bufs[0])
        tok_wait(2)
        issue(nd - 2, bufs[0], 2)
        wait(bufs[1])
        compute(nd - 3, bufs[1])
        tok_wait(3)
        issue(nd - 1, bufs[1], 3)
        wait(bufs[0])
        compute(nd - 2, bufs[0])
        wait(bufs[1])
        compute(nd - 1, bufs[1])

        pltpu.sync_copy(emb_v, emb_hbm.at[pl.ds(base, docs_per_w)])
        pltpu.sync_copy(abs_v, abs_hbm.at[pl.ds(base, docs_per_w)])

    emb, abssum = sc_kernel(token_ids, idf, W, b)
    embedding = emb.reshape(Bn, 1, Dn)
    padding_mask = (jnp.sum(abssum, axis=1, keepdims=True) == 0.0)
    return (embedding, padding_mask)


# speedup vs baseline: 1.0360x; 1.0360x over previous
import functools

import jax
import jax.numpy as jnp
from jax import lax
from jax.experimental import pallas as pl
from jax.experimental.pallas import tpu as pltpu
from jax.experimental.pallas import tpu_sc as plsc


def _sc_geometry():
    try:
        info = plsc.get_sparse_core_info()
        return info.num_cores, info.num_subcores, info.num_lanes
    except Exception:
        return 2, 16, 16


def kernel(token_ids, idf, W, b):
    Bn, Ln = token_ids.shape
    Vn, Dn = W.shape
    NC, NS, LANES = _sc_geometry()
    NW = NC * NS
    assert Bn % NW == 0
    docs_per_w = Bn // NW
    assert docs_per_w % 2 == 0
    assert Ln % 8 == 0
    n_chunks = Dn // LANES
    c0 = min(Ln, 104)
    c1 = Ln - c0
    Ln_pad = ((Ln + LANES - 1) // LANES) * LANES
    n_groups = Ln_pad // LANES

    mesh = plsc.VectorSubcoreMesh(core_axis_name="c", subcore_axis_name="s")

    @functools.partial(
        pl.kernel,
        mesh=mesh,
        out_type=[
            jax.ShapeDtypeStruct((Bn, Dn), jnp.float32),
            jax.ShapeDtypeStruct((Bn, LANES), jnp.float32)
        ],
        scratch_types=[
            pltpu.VMEM((docs_per_w * Ln,), jnp.int32),
            pltpu.VMEM((Ln_pad,), jnp.float32),
            pltpu.VMEM((Ln_pad,), jnp.float32),
            pltpu.VMEM((Ln_pad, Dn), jnp.float32),
            pltpu.VMEM((Ln_pad, Dn), jnp.float32),
            pltpu.VMEM((Dn,), jnp.float32),
            pltpu.VMEM((docs_per_w, Dn), jnp.float32),
            pltpu.VMEM((docs_per_w, LANES), jnp.float32),
            pltpu.SemaphoreType.DMA,
            pltpu.SemaphoreType.DMA,
        ],
    )
    def sc_kernel(tok_hbm, idf_hbm, w_hbm, b_hbm, emb_hbm, abs_hbm,
                  tok_v, idf0, idf1, rows0, rows1, b_v, emb_v, abs_v,
                  sem0, sem1):
        wid = lax.axis_index("s") * NC + lax.axis_index("c")
        base = wid * docs_per_w
        tok_base = pl.multiple_of(base * Ln, 8)
        pltpu.sync_copy(tok_hbm.at[pl.ds(tok_base, docs_per_w * Ln)], tok_v)
        pltpu.sync_copy(b_hbm, b_v)
        inv_l = 1.0 / float(Ln)
        bufs = ((idf0, rows0, sem0), (idf1, rows1, sem1))

        zvec = jnp.zeros((LANES,), jnp.float32)
        if Ln_pad > Ln:
            g_last = Ln_pad - LANES
            for idfb, rowb, _ in bufs:
                idfb[pl.ds(g_last, LANES)] = zvec
                for r in range(Ln, Ln_pad):
                    for c in range(n_chunks):
                        rowb[r, pl.ds(c * LANES, LANES)] = zvec

        def issue(d, buf):
            idfb, rowb, sem = buf
            off = pl.multiple_of(d * Ln, 8)
            i1 = tok_v.at[pl.ds(off, c0)]
            i2 = tok_v.at[pl.ds(off + c0, c1)]
            pltpu.async_copy(idf_hbm.at[i1], idfb.at[pl.ds(0, c0)], sem)
            pltpu.async_copy(idf_hbm.at[i2], idfb.at[pl.ds(c0, c1)], sem)
            pltpu.async_copy(w_hbm.at[i1], rowb.at[pl.ds(0, c0)], sem)
            pltpu.async_copy(w_hbm.at[i2], rowb.at[pl.ds(c0, c1)], sem)

        def wait(buf):
            idfb, rowb, sem = buf
            pltpu.make_async_copy(idf_hbm.at[pl.ds(0, c0)],
                                  idfb.at[pl.ds(0, c0)], sem).wait()
            pltpu.make_async_copy(idf_hbm.at[pl.ds(0, c1)],
                                  idfb.at[pl.ds(c0, c1)], sem).wait()
            pltpu.make_async_copy(w_hbm.at[pl.ds(0, c0)],
                                  rowb.at[pl.ds(0, c0)], sem).wait()
            pltpu.make_async_copy(w_hbm.at[pl.ds(0, c1)],
                                  rowb.at[pl.ds(c0, c1)], sem).wait()

        def compute(d, buf):
            idfb, rowb, _ = buf

            def grp_body(g, accs):
                tok0 = g * LANES
                wv = idfb[pl.ds(tok0, LANES)]
                for j in range(LANES):
                    wgt = wv[j]
                    accs = tuple(
                        accs[c] + wgt * rowb[tok0 + j,
                                             pl.ds(c * LANES, LANES)]
                        for c in range(n_chunks)
                    )
                return accs

            accs0 = tuple(jnp.zeros((LANES,), jnp.float32)
                          for _ in range(n_chunks))
            accs = lax.fori_loop(0, n_groups, grp_body, accs0)

            abssum = jnp.zeros((LANES,), jnp.float32)
            for c in range(n_chunks):
                e = accs[c] * inv_l + b_v[pl.ds(c * LANES, LANES)]
                emb_v[d, pl.ds(c * LANES, LANES)] = e
                abssum = abssum + jnp.abs(e)
            abs_v[d] = abssum

        issue(0, bufs[0])
        issue(1, bufs[1])

        def pipe_body(g, carry):
            d = 2 * g
            wait(bufs[0])
            compute(d, bufs[0])
            issue(d + 2, bufs[0])
            wait(bufs[1])
            compute(d + 1, bufs[1])
            issue(d + 3, bufs[1])
            return carry

        lax.fori_loop(0, docs_per_w // 2 - 1, pipe_body, 0)
        wait(bufs[0])
        compute(docs_per_w - 2, bufs[0])
        wait(bufs[1])
        compute(docs_per_w - 1, bufs[1])

        pltpu.sync_copy(emb_v, emb_hbm.at[pl.ds(base, docs_per_w)])
        pltpu.sync_copy(abs_v, abs_hbm.at[pl.ds(base, docs_per_w)])

    emb, abssum = sc_kernel(token_ids.reshape(-1), idf, W, b)
    embedding = emb.reshape(Bn, 1, Dn)
    padding_mask = (jnp.sum(abssum, axis=1, keepdims=True) == 0.0)
    return (embedding, padding_mask)


# speedup vs baseline: 1.1098x; 1.0712x over previous
import functools

import jax
import jax.numpy as jnp
from jax import lax
from jax.experimental import pallas as pl
from jax.experimental.pallas import tpu as pltpu
from jax.experimental.pallas import tpu_sc as plsc


def _sc_geometry():
    try:
        info = plsc.get_sparse_core_info()
        return info.num_cores, info.num_subcores, info.num_lanes
    except Exception:
        return 2, 16, 16


def kernel(token_ids, idf, W, b):
    Bn, Ln = token_ids.shape
    Vn, Dn = W.shape
    NC, NS, LANES = _sc_geometry()
    NW = NC * NS
    assert Bn % NW == 0
    docs_per_w = Bn // NW
    assert docs_per_w % 4 == 0 and docs_per_w >= 8
    assert Ln % 8 == 0
    n_chunks = Dn // LANES
    c0 = min(Ln, 104)
    c1 = Ln - c0
    Ln_pad = ((Ln + LANES - 1) // LANES) * LANES
    n_groups = Ln_pad // LANES

    mesh = plsc.VectorSubcoreMesh(core_axis_name="c", subcore_axis_name="s")

    @functools.partial(
        pl.kernel,
        mesh=mesh,
        out_type=[
            jax.ShapeDtypeStruct((Bn, Dn), jnp.float32),
            jax.ShapeDtypeStruct((Bn, LANES), jnp.float32)
        ],
        scratch_types=[
            pltpu.VMEM((docs_per_w * Ln,), jnp.int32),
            pltpu.VMEM((Ln_pad,), jnp.float32),
            pltpu.VMEM((Ln_pad,), jnp.float32),
            pltpu.VMEM((Ln_pad,), jnp.float32),
            pltpu.VMEM((Ln_pad,), jnp.float32),
            pltpu.VMEM((Ln_pad, Dn), jnp.float32),
            pltpu.VMEM((Ln_pad, Dn), jnp.float32),
            pltpu.VMEM((Ln_pad, Dn), jnp.float32),
            pltpu.VMEM((Ln_pad, Dn), jnp.float32),
            pltpu.VMEM((Dn,), jnp.float32),
            pltpu.VMEM((docs_per_w, Dn), jnp.float32),
            pltpu.VMEM((docs_per_w, LANES), jnp.float32),
            pltpu.SemaphoreType.DMA,
            pltpu.SemaphoreType.DMA,
            pltpu.SemaphoreType.DMA,
            pltpu.SemaphoreType.DMA,
        ],
    )
    def sc_kernel(tok_hbm, idf_hbm, w_hbm, b_hbm, emb_hbm, abs_hbm,
                  tok_v, idf0, idf1, idf2, idf3, rows0, rows1, rows2, rows3,
                  b_v, emb_v, abs_v, sem0, sem1, sem2, sem3):
        wid = lax.axis_index("s") * NC + lax.axis_index("c")
        base = wid * docs_per_w
        tok_base = pl.multiple_of(base * Ln, 8)
        pltpu.sync_copy(tok_hbm.at[pl.ds(tok_base, docs_per_w * Ln)], tok_v)
        pltpu.sync_copy(b_hbm, b_v)
        inv_l = 1.0 / float(Ln)
        bufs = ((idf0, rows0, sem0), (idf1, rows1, sem1),
                (idf2, rows2, sem2), (idf3, rows3, sem3))

        zvec = jnp.zeros((LANES,), jnp.float32)
        if Ln_pad > Ln:
            g_last = Ln_pad - LANES
            for idfb, rowb, _ in bufs:
                idfb[pl.ds(g_last, LANES)] = zvec
                for r in range(Ln, Ln_pad):
                    for c in range(n_chunks):
                        rowb[r, pl.ds(c * LANES, LANES)] = zvec

        def issue(d, buf):
            idfb, rowb, sem = buf
            off = pl.multiple_of(d * Ln, 8)
            i1 = tok_v.at[pl.ds(off, c0)]
            i2 = tok_v.at[pl.ds(off + c0, c1)]
            pltpu.async_copy(idf_hbm.at[i1], idfb.at[pl.ds(0, c0)], sem)
            pltpu.async_copy(idf_hbm.at[i2], idfb.at[pl.ds(c0, c1)], sem)
            pltpu.async_copy(w_hbm.at[i1], rowb.at[pl.ds(0, c0)], sem)
            pltpu.async_copy(w_hbm.at[i2], rowb.at[pl.ds(c0, c1)], sem)

        def wait(buf):
            idfb, rowb, sem = buf
            pltpu.make_async_copy(idf_hbm.at[pl.ds(0, c0)],
                                  idfb.at[pl.ds(0, c0)], sem).wait()
            pltpu.make_async_copy(idf_hbm.at[pl.ds(0, c1)],
                                  idfb.at[pl.ds(c0, c1)], sem).wait()
            pltpu.make_async_copy(w_hbm.at[pl.ds(0, c0)],
                                  rowb.at[pl.ds(0, c0)], sem).wait()
            pltpu.make_async_copy(w_hbm.at[pl.ds(0, c1)],
                                  rowb.at[pl.ds(c0, c1)], sem).wait()

        def compute(d, buf):
            idfb, rowb, _ = buf

            def grp_body(g, accs):
                tok0 = g * LANES
                wv = idfb[pl.ds(tok0, LANES)]
                for j in range(LANES):
                    wgt = wv[j]
                    accs = tuple(
                        accs[c] + wgt * rowb[tok0 + j,
                                             pl.ds(c * LANES, LANES)]
                        for c in range(n_chunks)
                    )
                return accs

            accs0 = tuple(jnp.zeros((LANES,), jnp.float32)
                          for _ in range(n_chunks))
            accs = lax.fori_loop(0, n_groups, grp_body, accs0)

            abssum = jnp.zeros((LANES,), jnp.float32)
            for c in range(n_chunks):
                e = accs[c] * inv_l + b_v[pl.ds(c * LANES, LANES)]
                emb_v[d, pl.ds(c * LANES, LANES)] = e
                abssum = abssum + jnp.abs(e)
            abs_v[d] = abssum

        issue(0, bufs[0])
        issue(1, bufs[1])
        issue(2, bufs[2])

        def pipe_body(g, carry):
            for j in range(4):
                d = 4 * g + j
                wait(bufs[j])
                compute(d, bufs[j])
                issue(d + 3, bufs[(j + 3) % 4])
            return carry

        lax.fori_loop(0, docs_per_w // 4 - 1, pipe_body, 0)

        nd = docs_per_w
        wait(bufs[0])
        compute(nd - 4, bufs[0])
        issue(nd - 1, bufs[3])
        wait(bufs[1])
        compute(nd - 3, bufs[1])
        wait(bufs[2])
        compute(nd - 2, bufs[2])
        wait(bufs[3])
        compute(nd - 1, bufs[3])

        pltpu.sync_copy(emb_v, emb_hbm.at[pl.ds(base, docs_per_w)])
        pltpu.sync_copy(abs_v, abs_hbm.at[pl.ds(base, docs_per_w)])

    emb, abssum = sc_kernel(token_ids.reshape(-1), idf, W, b)
    embedding = emb.reshape(Bn, 1, Dn)
    padding_mask = (jnp.sum(abssum, axis=1, keepdims=True) == 0.0)
    return (embedding, padding_mask)
